# baseline (speedup 1.0000x reference)
import jax
import jax.numpy as jnp
from jax.experimental import pallas as pl
from jax.experimental.pallas import tpu as pltpu


def _body(k_ref, tau_ref, x_ref, w_ref, b_ref, o_ref):
    ITERS = 10
    BR = x_ref.shape[0]
    D = x_ref.shape[1]
    k = k_ref[0]
    tau = tau_ref[0]
    xb = x_ref[...]
    xb16 = xb.astype(jnp.bfloat16)

    one_bf = jnp.bfloat16(1)
    zero_bf = jnp.bfloat16(0)
    ones_mx = jnp.ones((128, 128), jnp.bfloat16)
    kf = k.astype(jnp.float32)

    def masked_partials(vi, thr16, src=None):
        parts = []
        for j in range(D // 128):
            sl = vi[:, j * 128:(j + 1) * 128]
            mj = jnp.where(sl >= thr16, one_bf, zero_bf)
            if src is not None:
                mj = mj * src[:, j * 128:(j + 1) * 128]
            parts.append(mj)
        while len(parts) > 1:
            parts = [parts[j] + parts[j + 1] for j in range(0, len(parts), 2)]
        return parts[0]

    def decode16(enc32):
        e = enc32.astype(jnp.int16)
        tb = jnp.where(e >= 0, e, e ^ jnp.int16(0x7FFF))
        return jax.lax.bitcast_convert_type(tb, jnp.bfloat16).astype(jnp.float32)

    def row_scale(xh, xh16):
        R = xh.shape[0]
        b16 = jax.lax.bitcast_convert_type(xh16, jnp.int16)
        vi = jnp.where(b16 >= 0, b16, b16 ^ jnp.int16(0x7FFF))
        s1 = jnp.sum(xh, axis=1, keepdims=True)
        tsel = jnp.full((R, 1), -(1 << 15), jnp.int32)
        for i in range(ITERS):
            cand = tsel + jnp.int32(1 << (15 - i))
            p = masked_partials(vi, cand.astype(jnp.int16))
            c = jnp.dot(p, ones_mx, preferred_element_type=jnp.float32)
            tsel = jnp.where(c[:, :1] >= kf, cand, tsel)
        tval = decode16(tsel | jnp.int32((1 << (15 - ITERS)) if ITERS < 16 else 0))
        thr_enc = (tsel + jnp.int32(1 << (16 - ITERS))).astype(jnp.int16)
        cg = jnp.dot(masked_partials(vi, thr_enc), ones_mx,
                     preferred_element_type=jnp.float32)[:, :1]
        sx = jnp.dot(masked_partials(vi, thr_enc, xh16), ones_mx,
                     preferred_element_type=jnp.float32)[:, :1]
        s2 = sx + tval * (kf - cg)
        return jnp.exp(s1 / s2) / tau

    scale = row_scale(xb, xb16)

    y = jnp.dot(xb16, w_ref[...], preferred_element_type=jnp.float32)
    o_ref[...] = y * scale + b_ref[...]


def kernel(x, W, b, percentile, tau):
    B, D = x.shape
    C = W.shape[1]
    BR = min(1024, B)
    k_arr = (D - jnp.round(D * percentile / 100.0)).astype(jnp.int32).reshape(1)
    tau_arr = jnp.asarray(tau, jnp.float32).reshape(1)
    grid = (B // BR,)
    return pl.pallas_call(
        _body,
        grid=grid,
        in_specs=[
            pl.BlockSpec(memory_space=pltpu.SMEM),
            pl.BlockSpec(memory_space=pltpu.SMEM),
            pl.BlockSpec((BR, D), lambda i: (i, 0)),
            pl.BlockSpec((D, C), lambda i: (0, 0)),
            pl.BlockSpec((1, C), lambda i: (0, 0)),
        ],
        out_specs=pl.BlockSpec((BR, C), lambda i: (i, 0)),
        out_shape=jax.ShapeDtypeStruct((B, C), jnp.float32),
    )(k_arr, tau_arr, x, W.astype(jnp.bfloat16), b.reshape(1, C))
